# baseline (speedup 1.0000x reference)
import functools

import jax
import jax.numpy as jnp
from jax import lax
from jax.experimental import pallas as pl
from jax.experimental.pallas import tpu as pltpu
from jax.experimental.pallas import tpu_sc as plsc

B = 16384
D = 32
PK = 128 // D
NC = 2
NS = 16
NW = NC * NS
BPW = B // NW
CH = 128
NCH = BPW // CH
L = 16


def _sc_diff_body(ut_hbm, table_hbm, ps_hbm, ns_hbm, pf_hbm, nf_hbm,
                  out_hbm, ps_v, ns_v, pf_v, nf_v, u_v,
                  p_b0, p_b1, n_b0, n_b1, diff_v, sem0, sem1):
    wid = lax.axis_index("s") * NC + lax.axis_index("c")
    base = wid * BPW

    pltpu.sync_copy(ps_hbm.at[wid], ps_v)
    pltpu.sync_copy(ns_hbm.at[wid], ns_v)
    pltpu.sync_copy(pf_hbm.at[wid], pf_v)
    pltpu.sync_copy(nf_hbm.at[wid], nf_v)
    pltpu.sync_copy(ut_hbm.at[:, pl.ds(base, BPW)], u_v)

    p_bufs = (p_b0, p_b1)
    n_bufs = (n_b0, n_b1)
    sems = (sem0, sem1)
    lane = lax.iota(jnp.int32, L)

    def fire(j):
        s = sems[j % 2]
        return (
            pltpu.async_copy(table_hbm.at[ps_v.at[j]], p_bufs[j % 2], s),
            pltpu.async_copy(table_hbm.at[ns_v.at[j]], n_bufs[j % 2], s),
        )

    inflight = fire(0)
    for j in range(NCH):
        nxt = fire(j + 1) if j + 1 < NCH else ()
        for c in inflight:
            c.wait()
        inflight = nxt

        p_v = p_bufs[j % 2]
        n_v = n_bufs[j % 2]

        def group(k, carry, p_v=p_v, n_v=n_v, j=j):
            off = j * CH + k * L
            b = off + lane
            rows = k * L + lane
            pbase = pf_v[pl.ds(off, L)] & 3
            nbase = nf_v[pl.ds(off, L)] & 3
            acc = jnp.zeros((L,), jnp.float32)
            for d in range(D):
                rot = (lane + d) & (D - 1)
                u = plsc.load_gather(u_v, [rot, b])
                p = plsc.load_gather(p_v, [rows, pbase + (rot << 2)])
                n = plsc.load_gather(n_v, [rows, nbase + (rot << 2)])
                acc = acc + u * (p - n)
            diff_v[pl.ds(off, L)] = acc
            return carry

        lax.fori_loop(0, CH // L, group, 0)

    pltpu.sync_copy(diff_v, out_hbm.at[pl.ds(base, BPW)])


_sc_diff = functools.partial(
    pl.kernel,
    mesh=plsc.VectorSubcoreMesh(core_axis_name="c", subcore_axis_name="s"),
    out_type=jax.ShapeDtypeStruct((B,), jnp.float32),
    scratch_types=[
        pltpu.VMEM((NCH, CH), jnp.int32),
        pltpu.VMEM((NCH, CH), jnp.int32),
        pltpu.VMEM((BPW,), jnp.int32),
        pltpu.VMEM((BPW,), jnp.int32),
        pltpu.VMEM((D, BPW), jnp.float32),
        pltpu.VMEM((CH, 128), jnp.float32),
        pltpu.VMEM((CH, 128), jnp.float32),
        pltpu.VMEM((CH, 128), jnp.float32),
        pltpu.VMEM((CH, 128), jnp.float32),
        pltpu.VMEM((BPW,), jnp.float32),
        pltpu.SemaphoreType.DMA,
        pltpu.SemaphoreType.DMA,
    ],
    compiler_params=pltpu.CompilerParams(
        needs_layout_passes=False,
    ),
)(_sc_diff_body)


NI = 1000000
PCH = 512
NPC = 999936 // PCH
PIT = (NPC + NW - 1) // NW


def _sc_pack_body(tt_hbm, tail_hbm, out_hbm, in_v, tail_v, out_v):
    wid = lax.axis_index("s") * NC + lax.axis_index("c")
    lane = lax.iota(jnp.int32, L)
    dvec = lambda m: 4 * m + (lane >> 2)
    gvec = lane & 3

    def chunk(k, carry):
        c = k * NW + wid

        @pl.when(c < NPC)
        def _():
            start = pl.multiple_of(c * PCH, 128)
            pltpu.sync_copy(tt_hbm.at[:, pl.ds(start, PCH)], in_v)

            def row(r4, carry2):
                for q in range(4):
                    r = r4 * 4 + q
                    col = 4 * r + gvec
                    for m in range(8):
                        out_v[r, pl.ds(m * L, L)] = plsc.load_gather(
                            in_v, [dvec(m), col])
                return carry2

            lax.fori_loop(0, PCH // 16, row, 0)
            pltpu.sync_copy(out_v, out_hbm.at[pl.ds(c * (PCH // 4), PCH // 4)])

        return carry

    lax.fori_loop(0, PIT, chunk, 0)

    @pl.when(wid == NW - 1)
    def _():
        pltpu.sync_copy(tail_hbm, tail_v)
        for r in range(16):
            li = 4 * r + gvec
            for m in range(8):
                out_v[r, pl.ds(m * L, L)] = plsc.load_gather(
                    tail_v, [li, dvec(m)])
        pltpu.sync_copy(out_v.at[pl.ds(0, 16)], out_hbm.at[pl.ds(249984, 16)])


_sc_pack = functools.partial(
    pl.kernel,
    mesh=plsc.VectorSubcoreMesh(core_axis_name="c", subcore_axis_name="s"),
    out_type=jax.ShapeDtypeStruct((NI // 4, 128), jnp.float32),
    scratch_types=[
        pltpu.VMEM((D, PCH), jnp.float32),
        pltpu.VMEM((64, D), jnp.float32),
        pltpu.VMEM((PCH // 4, 128), jnp.float32),
    ],
    compiler_params=pltpu.CompilerParams(
        needs_layout_passes=False,
    ),
)(_sc_pack_body)


def _tc_loss_body(x_ref, o_ref):
    x = x_ref[...]
    y = -x
    sp = jnp.maximum(y, 0.0) + jnp.log1p(jnp.exp(-jnp.abs(y)))
    o_ref[0, 0] = jnp.sum(sp) * jnp.float32(1.0 / B)


_tc_loss = pl.pallas_call(
    _tc_loss_body,
    out_shape=jax.ShapeDtypeStruct((1, 1), jnp.float32),
    out_specs=pl.BlockSpec(memory_space=pltpu.SMEM),
)


def kernel(user_embeddings, item_embeddings, positive_item_indices,
           negative_item_indices):
    ut = user_embeddings.T
    table_p = _sc_pack(item_embeddings.T, item_embeddings[999936:])
    pos = positive_item_indices.astype(jnp.int32)
    neg = negative_item_indices.astype(jnp.int32)
    ps = (pos >> 2).reshape(NW, NCH, CH)
    ns = (neg >> 2).reshape(NW, NCH, CH)
    pf = pos.reshape(NW, BPW)
    nf = neg.reshape(NW, BPW)
    diff = _sc_diff(ut, table_p, ps, ns, pf, nf)
    loss = _tc_loss(diff.reshape(B // 128, 128))
    return loss[0, 0]


# speedup vs baseline: 1.0737x; 1.0737x over previous
import functools

import jax
import jax.numpy as jnp
from jax import lax
from jax.experimental import pallas as pl
from jax.experimental.pallas import tpu as pltpu
from jax.experimental.pallas import tpu_sc as plsc

B = 16384
D = 32
PK = 128 // D
NC = 2
NS = 16
NW = NC * NS
BPW = B // NW
CH = 128
NCH = BPW // CH
L = 16


def _sc_diff_body(ut_hbm, table_hbm, ps_hbm, ns_hbm, pf_hbm, nf_hbm,
                  out_hbm, ps_v, ns_v, pf_v, nf_v, u_v,
                  p_b0, p_b1, n_b0, n_b1, diff_v, sem0, sem1):
    wid = lax.axis_index("s") * NC + lax.axis_index("c")
    base = wid * BPW

    pltpu.sync_copy(ps_hbm.at[wid], ps_v)
    pltpu.sync_copy(ns_hbm.at[wid], ns_v)
    pltpu.sync_copy(pf_hbm.at[wid], pf_v)
    pltpu.sync_copy(nf_hbm.at[wid], nf_v)
    pltpu.sync_copy(ut_hbm.at[:, pl.ds(base, BPW)], u_v)

    p_bufs = (p_b0, p_b1)
    n_bufs = (n_b0, n_b1)
    sems = (sem0, sem1)
    lane = lax.iota(jnp.int32, L)

    def fire(j):
        s = sems[j % 2]
        return (
            pltpu.async_copy(table_hbm.at[ps_v.at[j]], p_bufs[j % 2], s),
            pltpu.async_copy(table_hbm.at[ns_v.at[j]], n_bufs[j % 2], s),
        )

    inflight = fire(0)
    for j in range(NCH):
        nxt = fire(j + 1) if j + 1 < NCH else ()
        for c in inflight:
            c.wait()
        inflight = nxt

        p_v = p_bufs[j % 2]
        n_v = n_bufs[j % 2]

        def group(k, carry, p_v=p_v, n_v=n_v, j=j):
            off = j * CH + k * L
            b = off + lane
            rows = k * L + lane
            pbase = pf_v[pl.ds(off, L)] & 3
            nbase = nf_v[pl.ds(off, L)] & 3
            acc = jnp.zeros((L,), jnp.float32)
            for d in range(D):
                rot = (lane + d) & (D - 1)
                u = plsc.load_gather(u_v, [rot, b])
                p = plsc.load_gather(p_v, [rows, pbase + (rot << 2)])
                n = plsc.load_gather(n_v, [rows, nbase + (rot << 2)])
                acc = acc + u * (p - n)
            diff_v[pl.ds(off, L)] = acc
            return carry

        lax.fori_loop(0, CH // L, group, 0)

    pltpu.sync_copy(diff_v, out_hbm.at[pl.ds(base, BPW)])


_sc_diff = functools.partial(
    pl.kernel,
    mesh=plsc.VectorSubcoreMesh(core_axis_name="c", subcore_axis_name="s"),
    out_type=jax.ShapeDtypeStruct((B,), jnp.float32),
    scratch_types=[
        pltpu.VMEM((NCH, CH), jnp.int32),
        pltpu.VMEM((NCH, CH), jnp.int32),
        pltpu.VMEM((BPW,), jnp.int32),
        pltpu.VMEM((BPW,), jnp.int32),
        pltpu.VMEM((D, BPW), jnp.float32),
        pltpu.VMEM((CH, 128), jnp.float32),
        pltpu.VMEM((CH, 128), jnp.float32),
        pltpu.VMEM((CH, 128), jnp.float32),
        pltpu.VMEM((CH, 128), jnp.float32),
        pltpu.VMEM((BPW,), jnp.float32),
        pltpu.SemaphoreType.DMA,
        pltpu.SemaphoreType.DMA,
    ],
    compiler_params=pltpu.CompilerParams(
        needs_layout_passes=False,
    ),
)(_sc_diff_body)


NI = 1000000
PCH = 512
NPC = 999936 // PCH
PIT = (NPC + NW - 1) // NW


def _sc_pack_body(tt_hbm, tail_hbm, out_hbm, in_v, in2_v, tail_v, out_v, psem):
    wid = lax.axis_index("s") * NC + lax.axis_index("c")
    lane = lax.iota(jnp.int32, L)
    dvec = lambda m: 4 * m + (lane >> 2)
    gvec = lane & 3

    def chunk(k, carry):
        c = k * NW + wid

        @pl.when(c < NPC)
        def _():
            start = pl.multiple_of(c * PCH, 128)
            pltpu.sync_copy(tt_hbm.at[:, pl.ds(start, PCH)], in_v)
            for d_ in range(D):
                off = 4 * (d_ % 4)
                for t_ in range(PCH // L):
                    in2_v[d_, pl.ds(off + t_ * L, L)] = (
                        in_v[d_, pl.ds(t_ * L, L)])

            def row(r4, carry2):
                for q in range(4):
                    r = r4 * 4 + q
                    col = 4 * r + lane
                    for m in range(8):
                        out_v[r, pl.ds(m * L, L)] = plsc.load_gather(
                            in2_v, [dvec(m), col])
                return carry2

            lax.fori_loop(0, PCH // 16, row, 0)
            pltpu.sync_copy(out_v, out_hbm.at[pl.ds(c * (PCH // 4), PCH // 4)])

        return carry

    lax.fori_loop(0, PIT, chunk, 0)

    @pl.when(wid == NW - 1)
    def _():
        pltpu.sync_copy(tail_hbm, tail_v)
        for r in range(16):
            li = 4 * r + gvec
            for m in range(8):
                out_v[r, pl.ds(m * L, L)] = plsc.load_gather(
                    tail_v, [li, dvec(m)])
        pltpu.sync_copy(out_v.at[pl.ds(0, 16)], out_hbm.at[pl.ds(249984, 16)])


_sc_pack = functools.partial(
    pl.kernel,
    mesh=plsc.VectorSubcoreMesh(core_axis_name="c", subcore_axis_name="s"),
    out_type=jax.ShapeDtypeStruct((NI // 4, 128), jnp.float32),
    scratch_types=[
        pltpu.VMEM((D, PCH), jnp.float32),
        pltpu.VMEM((D, PCH + 16), jnp.float32),
        pltpu.VMEM((64, D), jnp.float32),
        pltpu.VMEM((PCH // 4, 128), jnp.float32),
        pltpu.SemaphoreType.DMA,
    ],
    compiler_params=pltpu.CompilerParams(
        needs_layout_passes=False,
    ),
)(_sc_pack_body)


def _tc_loss_body(x_ref, o_ref):
    x = x_ref[...]
    y = -x
    sp = jnp.maximum(y, 0.0) + jnp.log1p(jnp.exp(-jnp.abs(y)))
    o_ref[0, 0] = jnp.sum(sp) * jnp.float32(1.0 / B)


_tc_loss = pl.pallas_call(
    _tc_loss_body,
    out_shape=jax.ShapeDtypeStruct((1, 1), jnp.float32),
    out_specs=pl.BlockSpec(memory_space=pltpu.SMEM),
)


def kernel(user_embeddings, item_embeddings, positive_item_indices,
           negative_item_indices):
    ut = user_embeddings.T
    table_p = _sc_pack(item_embeddings.T, item_embeddings[999936:])
    pos = positive_item_indices.astype(jnp.int32)
    neg = negative_item_indices.astype(jnp.int32)
    ps = (pos >> 2).reshape(NW, NCH, CH)
    ns = (neg >> 2).reshape(NW, NCH, CH)
    pf = pos.reshape(NW, BPW)
    nf = neg.reshape(NW, BPW)
    diff = _sc_diff(ut, table_p, ps, ns, pf, nf)
    loss = _tc_loss(diff.reshape(B // 128, 128))
    return loss[0, 0]
